# baseline (speedup 1.0000x reference)
import functools

import jax
import jax.numpy as jnp
from jax import lax
from jax.experimental import pallas as pl
from jax.experimental.pallas import tpu as pltpu
from jax.experimental.pallas import tpu_sc as plsc

NUM_EMB = 1000000
DIM = 64
NC = 2
NS = 16
NW = NC * NS
CHUNK = 128


@functools.partial(jax.jit, static_argnums=(2, 3))
def _gather_call(x_r, table, n_chunks, b_per_w):
    mesh = plsc.VectorSubcoreMesh(core_axis_name="c", subcore_axis_name="s")

    @functools.partial(
        pl.kernel,
        mesh=mesh,
        out_type=jax.ShapeDtypeStruct((NW, b_per_w, DIM), jnp.float32),
        scratch_types=[
            pltpu.VMEM((n_chunks, CHUNK), jnp.int32),
            pltpu.VMEM((CHUNK, DIM), jnp.float32),
            pltpu.SemaphoreType.DMA,
        ],
        compiler_params=pltpu.CompilerParams(use_tc_tiling_on_sc=False),
    )
    def body(x_hbm, table_hbm, out_hbm, idx_v, rows_v, sem):
        wid = lax.axis_index("s") * NC + lax.axis_index("c")
        pltpu.sync_copy(x_hbm.at[wid], idx_v)

        def step(j, _):
            pltpu.async_copy(table_hbm.at[idx_v.at[j]], rows_v, sem).wait()
            pltpu.sync_copy(rows_v, out_hbm.at[wid, pl.ds(j * CHUNK, CHUNK)])
            return _

        lax.fori_loop(0, n_chunks, step, None)

    return body(x_r, table)


def kernel(x, table):
    b, s = x.shape
    total = b * s
    b_per_w = total // NW
    n_chunks = b_per_w // CHUNK
    x_r = x.astype(jnp.int32).reshape(NW, n_chunks, CHUNK)
    out = _gather_call(x_r, table, n_chunks, b_per_w)
    return out.reshape(b, s, DIM)


# speedup vs baseline: 1.2999x; 1.2999x over previous
import functools

import jax
import jax.numpy as jnp
from jax import lax
from jax.experimental import pallas as pl
from jax.experimental.pallas import tpu as pltpu
from jax.experimental.pallas import tpu_sc as plsc

NC = 2
NS = 16
NW = NC * NS
NB = 8


@functools.partial(jax.jit, static_argnums=(2, 3, 4))
def _gather_call(x, table, bsz, seq, dim):
    b_per_w = bsz // NW
    n_chunks = b_per_w // NB
    rows_per_chunk = NB * seq
    mesh = plsc.VectorSubcoreMesh(core_axis_name="c", subcore_axis_name="s")

    @functools.partial(
        pl.kernel,
        mesh=mesh,
        out_type=jax.ShapeDtypeStruct((bsz, seq, dim), jnp.float32),
        scratch_types=[
            pltpu.VMEM((NB, seq), jnp.int32),
            pltpu.VMEM((NB, seq, dim), jnp.float32),
            pltpu.SemaphoreType.DMA,
        ],
    )
    def body(x_hbm, table_hbm, out_hbm, idx_v, buf, sem):
        wid = lax.axis_index("s") * NC + lax.axis_index("c")

        def chunk(c, _):
            b0 = pl.multiple_of(wid * b_per_w + c * NB, NB)
            pltpu.sync_copy(x_hbm.at[pl.ds(b0, NB), :], idx_v)
            for jb in range(NB):
                va = idx_v[jb, pl.ds(0, 16)]
                vb = idx_v[jb, pl.ds(seq - 16, 16)]
                for s in range(seq):
                    i = va[s] if s < 16 else vb[s - (seq - 16)]
                    pltpu.async_copy(table_hbm.at[i], buf.at[jb, s], sem)

            def drain(t, _):
                pltpu.make_async_copy(table_hbm.at[0], buf.at[0, 0], sem).wait()
                return _

            lax.fori_loop(0, rows_per_chunk, drain, None)
            pltpu.sync_copy(buf, out_hbm.at[pl.ds(b0, NB)])
            return _

        lax.fori_loop(0, n_chunks, chunk, None)

    return body(x, table)


def kernel(x, table):
    bsz, seq = x.shape
    return _gather_call(x.astype(jnp.int32), table, bsz, seq, table.shape[1])


# speedup vs baseline: 1.7841x; 1.3725x over previous
import functools

import jax
import jax.numpy as jnp
from jax import lax
from jax.experimental import pallas as pl
from jax.experimental.pallas import tpu as pltpu
from jax.experimental.pallas import tpu_sc as plsc

NC = 2
NS = 16
NW = NC * NS
NB = 8


@functools.partial(jax.jit, static_argnums=(2, 3, 4))
def _gather_call(x, table, bsz, seq, dim):
    b_per_w = bsz // NW
    n_chunks = b_per_w // NB
    rows_per_chunk = NB * seq
    row_w = seq * dim
    mesh = plsc.VectorSubcoreMesh(core_axis_name="c", subcore_axis_name="s")

    @functools.partial(
        pl.kernel,
        mesh=mesh,
        out_type=jax.ShapeDtypeStruct((bsz, row_w), jnp.float32),
        scratch_types=[
            pltpu.VMEM((2, NB, seq), jnp.int32),
            pltpu.VMEM((2, NB, row_w), jnp.float32),
            pltpu.SemaphoreType.DMA,
            pltpu.SemaphoreType.DMA,
            pltpu.SemaphoreType.DMA,
        ],
    )
    def body(x_hbm, table_hbm, out_hbm, idx_v, buf, gsem, isem, osem):
        wid = lax.axis_index("s") * NC + lax.axis_index("c")
        base = pl.multiple_of(wid * b_per_w, NB)

        pltpu.async_copy(x_hbm.at[pl.ds(base, NB), :], idx_v.at[0], isem)

        def chunk(c, _):
            par = c % 2
            b0 = pl.multiple_of(base + c * NB, NB)
            @pl.when(c + 1 < n_chunks)
            def _():
                pltpu.async_copy(
                    x_hbm.at[pl.ds(b0 + NB, NB), :], idx_v.at[1 - par], isem
                )

            pltpu.make_async_copy(
                x_hbm.at[pl.ds(0, NB), :], idx_v.at[par], isem
            ).wait()

            @pl.when(c >= 2)
            def _():
                pltpu.make_async_copy(
                    buf.at[par], out_hbm.at[pl.ds(0, NB)], osem
                ).wait()

            for jb in range(NB):
                va = idx_v[par, jb, pl.ds(0, 16)]
                vb = idx_v[par, jb, pl.ds(seq - 16, 16)]
                for s in range(seq):
                    i = va[s] if s < 16 else vb[s - (seq - 16)]
                    pltpu.async_copy(
                        table_hbm.at[i], buf.at[par, jb, pl.ds(s * dim, dim)],
                        gsem,
                    )

            pltpu.make_async_copy(
                out_hbm.at[pl.ds(0, NB)], buf.at[par], gsem
            ).wait()
            pltpu.async_copy(buf.at[par], out_hbm.at[pl.ds(b0, NB)], osem)
            return _

        lax.fori_loop(0, n_chunks, chunk, None)
        pltpu.make_async_copy(buf.at[0], out_hbm.at[pl.ds(0, NB)], osem).wait()
        pltpu.make_async_copy(buf.at[0], out_hbm.at[pl.ds(0, NB)], osem).wait()

    return body(x, table)


def kernel(x, table):
    bsz, seq = x.shape
    dim = table.shape[1]
    out2 = _gather_call(x.astype(jnp.int32), table, bsz, seq, dim)
    return out2.reshape(bsz, seq, dim)
